# baseline (speedup 1.0000x reference)
import functools

import jax
import jax.numpy as jnp
from jax import lax
from jax.experimental import pallas as pl
from jax.experimental.pallas import tpu as pltpu
from jax.experimental.pallas import tpu_sc as plsc

NC, NS, LANES = 2, 16, 16
NW = NC * NS
B, L = 16, 4096
ELEMS = B * L * 2
CHUNK = ELEMS // NW
EPD = 2 * L


def _i32(x):
    return jnp.int32(x)


def _f32(x):
    return jnp.float32(x)


def _logf(v):
    iv = lax.bitcast_convert_type(v, jnp.int32)
    e = (iv >> _i32(23)) - _i32(126)
    m = lax.bitcast_convert_type((iv & _i32(0x007FFFFF)) | _i32(0x3F000000),
                                 jnp.float32)
    c = m < _f32(0.70710678)
    e = e - jnp.where(c, _i32(1), _i32(0))
    ef = e.astype(jnp.float32)
    x = jnp.where(c, m + m, m) - _f32(1.0)
    z = x * x
    p = _f32(7.0376836292e-2)
    for coef in (-1.1514610310e-1, 1.1676998740e-1, -1.2420140846e-1,
                 1.4249322787e-1, -1.6668057665e-1, 2.0000714765e-1,
                 -2.4999993993e-1, 3.3333331174e-1):
        p = p * x + _f32(coef)
    y = p * x * z
    y = y + ef * _f32(-2.12194440e-4)
    y = y - _f32(0.5) * z
    return x + y + ef * _f32(0.693359375)


@functools.partial(
    pl.kernel,
    mesh=plsc.VectorSubcoreMesh(core_axis_name="c", subcore_axis_name="s"),
    out_type=jax.ShapeDtypeStruct((NW, LANES), jnp.float32),
    scratch_types=[
        pltpu.VMEM((CHUNK,), jnp.float32),
        pltpu.VMEM((CHUNK,), jnp.float32),
        pltpu.VMEM((LANES,), jnp.int32),
        pltpu.VMEM((LANES,), jnp.int32),
        pltpu.VMEM((LANES,), jnp.float32),
        pltpu.SemaphoreType.DMA,
    ],
)
def _sc_masked_logsum(yt_hbm, yp_hbm, dl_hbm, out_hbm, yt_v, yp_v, idx_v,
                      dlb_v, res_v, sem):
    wid = lax.axis_index("s") * NC + lax.axis_index("c")
    base = wid * _i32(CHUNK)
    pltpu.sync_copy(yt_hbm.at[pl.ds(base, CHUNK)], yt_v)
    pltpu.sync_copy(yp_hbm.at[pl.ds(base, CHUNK)], yp_v)

    doc = wid >> _i32(1)
    tok0 = (wid & _i32(1)) * _i32(CHUNK // 2)
    idx_v[...] = jnp.full((LANES,), doc, jnp.int32)
    pltpu.async_copy(dl_hbm.at[idx_v], dlb_v, sem).wait()
    dlb = dlb_v[...]
    half_iota = lax.iota(jnp.int32, LANES) >> _i32(1)

    def step(k, acc):
        off = k * _i32(LANES)
        yp = yp_v[pl.ds(off, LANES)]
        yt = yt_v[pl.ds(off, LANES)]
        tok = (tok0 + (off >> _i32(1))) + half_iota
        w = jnp.where(tok < dlb, yt, _f32(0.0))
        return acc + _logf(yp) * w

    acc = lax.fori_loop(_i32(0), _i32(CHUNK // LANES), step,
                        jnp.zeros((LANES,), jnp.float32))
    res_v[...] = acc
    pltpu.sync_copy(res_v, out_hbm.at[wid])


def kernel(y_true, y_pred, doc_len):
    yt = y_true.reshape(-1)
    yp = y_pred.reshape(-1)
    dl = doc_len.astype(jnp.int32)
    partial = _sc_masked_logsum(yt, yp, dl)
    total = jnp.sum(partial.astype(jnp.float64))
    denom = jnp.sum(doc_len).astype(jnp.float64)
    return -total / denom
